# baseline (speedup 1.0000x reference)
import functools

import jax
import jax.numpy as jnp
from jax import lax
from jax.experimental import pallas as pl
from jax.experimental.pallas import tpu as pltpu
from jax.experimental.pallas import tpu_sc as plsc

_CHUNK = 128
_KFIRE = 8


@functools.cache
def _build(B, D):
    info = plsc.get_sparse_core_info()
    NC, NS = info.num_cores, info.num_subcores
    NW = NC * NS
    n_per_w = B // NW
    round_n = _CHUNK * _KFIRE
    n_rounds = n_per_w // round_n
    assert B % NW == 0 and n_per_w % round_n == 0
    n_slots = n_rounds + 2

    mesh = plsc.VectorSubcoreMesh(core_axis_name="c", subcore_axis_name="s")

    @functools.partial(
        pl.kernel,
        mesh=mesh,
        compiler_params=pltpu.CompilerParams(
            use_tc_tiling_on_sc=False, needs_layout_passes=False
        ),
        out_type=jax.ShapeDtypeStruct((B, D), jnp.float32),
        scratch_types=[
            pltpu.VMEM((n_per_w,), jnp.int32),
            pltpu.VMEM((round_n, D), jnp.float32),
            pltpu.VMEM((round_n, D), jnp.float32),
            pltpu.SemaphoreType.DMA,
            pltpu.SemaphoreType.DMA,
            pltpu.SemaphoreType.DMA,
            pltpu.SemaphoreType.DMA,
        ],
    )
    def emb_kernel(idx_hbm, table_hbm, out_hbm, idx_v, r0, r1, g0, g1, s0, s1):
        rows = (r0, r1)
        gsem = (g0, g1)
        ssem = (s0, s1)
        wid = lax.axis_index("s") * NC + lax.axis_index("c")
        base = wid * n_per_w

        pltpu.sync_copy(idx_hbm.at[pl.ds(base, n_per_w)], idx_v)

        def gather_descs(r, h):
            ds = []
            for b in range(_KFIRE):
                src = table_hbm.at[idx_v.at[pl.ds(r * round_n + b * _CHUNK, _CHUNK)]]
                dst = rows[h].at[pl.ds(b * _CHUNK, _CHUNK)]
                ds.append((src, dst))
            return ds

        def fire_gathers(r, h):
            for src, dst in gather_descs(r, h):
                pltpu.async_copy(src, dst, gsem[h])

        def drain_gathers(r, h):
            for src, dst in gather_descs(r, h):
                pltpu.make_async_copy(src, dst, gsem[h]).wait()

        def store_desc(r, h):
            return rows[h], out_hbm.at[pl.ds(base + r * round_n, round_n)], ssem[h]

        def slot(r, h):
            @pl.when((r >= 1) & (r <= n_rounds))
            def _ship_prev():
                drain_gathers(r - 1, 1 - h)
                src, dst, sem = store_desc(r - 1, 1 - h)
                pltpu.async_copy(src, dst, sem)

            @pl.when((r >= 2) & (r <= n_rounds + 1))
            def _free_buf():
                src, dst, sem = store_desc(r - 2, h)
                pltpu.make_async_copy(src, dst, sem).wait()

            @pl.when(r < n_rounds)
            def _fetch():
                fire_gathers(r, h)

        def body(rr, carry):
            slot(2 * rr, 0)
            slot(2 * rr + 1, 1)
            return carry

        lax.fori_loop(0, (n_slots + 1) // 2, body, 0)

    return emb_kernel


def kernel(inputs, table):
    S0, S1 = inputs.shape
    D = table.shape[1]
    idx = inputs.reshape(S0 * S1).astype(jnp.int32)
    out = _build(S0 * S1, D)(idx, table)
    return out.reshape(S0, S1, D)


# speedup vs baseline: 1.7363x; 1.7363x over previous
import functools

import jax
import jax.numpy as jnp
from jax import lax
from jax.experimental import pallas as pl
from jax.experimental.pallas import tpu as pltpu
from jax.experimental.pallas import tpu_sc as plsc


@functools.cache
def _build(S0, S1, D):
    B = S0 * S1
    info = plsc.get_sparse_core_info()
    NC, NS, L = info.num_cores, info.num_subcores, info.num_lanes
    NW = NC * NS
    assert S0 % (128 * NW) == 0 and D % 8 == 0 and L == 16
    b_per_w = S0 // NW
    nbl = b_per_w // 128
    n_units = S1 * nbl
    n_cc = D // 8

    mesh = plsc.VectorSubcoreMesh(core_axis_name="c", subcore_axis_name="s")

    @functools.partial(
        pl.kernel,
        mesh=mesh,
        compiler_params=pltpu.CompilerParams(
            use_tc_tiling_on_sc=False, needs_layout_passes=False
        ),
        out_type=jax.ShapeDtypeStruct((S1, n_cc, S0 // 128, 8, 128), jnp.float32),
        scratch_types=[
            pltpu.VMEM((b_per_w * S1,), jnp.int32),
            pltpu.VMEM((128,), jnp.int32),
            pltpu.VMEM((128,), jnp.int32),
            pltpu.VMEM((128, D), jnp.float32),
            pltpu.VMEM((128, D), jnp.float32),
            pltpu.VMEM((D, 128), jnp.float32),
            pltpu.VMEM((D, 128), jnp.float32),
            pltpu.SemaphoreType.DMA,
            pltpu.SemaphoreType.DMA,
            pltpu.SemaphoreType.DMA,
            pltpu.SemaphoreType.DMA,
        ],
    )
    def emb_kernel(idx_hbm, table_hbm, out5, idx_v, u0, u1, r0, r1, t0, t1,
                   sg0, sg1, ss0, ss1):
        uidx = (u0, u1)
        rows = (r0, r1)
        trs = (t0, t1)
        sg = (sg0, sg1)
        ss = (ss0, ss1)
        wid = lax.axis_index("s") * NC + lax.axis_index("c")

        pltpu.sync_copy(idx_hbm.at[pl.ds(wid * b_per_w * S1, b_per_w * S1)], idx_v)

        iota = lax.broadcasted_iota(jnp.int32, (L,), 0)

        def unit_tb(u):
            return u // nbl, u % nbl

        def compact_idx(u, b):
            t, bl = unit_tb(u)
            for k in range(128 // L):
                p = (128 * bl + L * k + iota) * S1 + t
                uidx[b][pl.ds(L * k, L)] = plsc.load_gather(idx_v, [p])

        def gather_start(u, b):
            pltpu.async_copy(table_hbm.at[uidx[b]], rows[b], sg[b])

        def gather_wait(u, b):
            pltpu.make_async_copy(table_hbm.at[uidx[b]], rows[b], sg[b]).wait()

        def transpose(u, b):
            for kb in range(128 // L):
                rvec = L * kb + iota
                for col in range(D):
                    vals = plsc.load_gather(rows[b], [rvec, iota * 0 + col])
                    trs[b][col, pl.ds(L * kb, L)] = vals

        def store_start(u, b):
            t, bl = unit_tb(u)
            bhi = wid * nbl + bl
            for cc in range(n_cc):
                pltpu.async_copy(
                    trs[b].at[pl.ds(8 * cc, 8)], out5.at[t, cc, bhi], ss[b]
                )

        def store_wait(u, b):
            t, bl = unit_tb(u)
            bhi = wid * nbl + bl
            for cc in range(n_cc):
                pltpu.make_async_copy(
                    trs[b].at[pl.ds(8 * cc, 8)], out5.at[t, cc, bhi], ss[b]
                ).wait()

        compact_idx(0, 0)
        gather_start(0, 0)

        def body(u, carry):
            b = lax.rem(u, 2)

            @pl.when(b == 0)
            def _even():
                compact_idx(u + 1, 1)
                gather_start(u + 1, 1)
                gather_wait(u, 0)

                @pl.when(u >= 2)
                def _():
                    store_wait(u - 2, 0)

                transpose(u, 0)
                store_start(u, 0)

            @pl.when(b == 1)
            def _odd():
                @pl.when(u + 1 < n_units)
                def _():
                    compact_idx(u + 1, 0)
                    gather_start(u + 1, 0)

                gather_wait(u, 1)

                @pl.when(u >= 2)
                def _():
                    store_wait(u - 2, 1)

                transpose(u, 1)
                store_start(u, 1)

            return carry

        lax.fori_loop(0, n_units, body, 0)
        store_wait(n_units - 2, 0)
        store_wait(n_units - 1, 1)

    return emb_kernel


def kernel(inputs, table):
    S0, S1 = inputs.shape
    D = table.shape[1]
    B = S0 * S1
    idx = inputs.reshape(B).astype(jnp.int32)
    out5 = _build(S0, S1, D)(idx, table)
    return out5.transpose(2, 4, 0, 1, 3).reshape(S0, S1, D)
